# baseline (device time: 358829 ns/iter reference)
import jax
import jax.numpy as jnp
from jax import lax
from jax.experimental import pallas as pl
from jax.experimental.pallas import tpu as pltpu

P = 16
COMM_DTYPE = jnp.bfloat16
NRINGS = 8


def kernel(x, w_mat, scale_x, scale_w):
    m_full, _k_shard = x.shape
    _, n = w_mat.shape
    m_chunk = m_full // P
    ns = n // NRINGS
    ring_cfg = [(ri < NRINGS // 2, ri * ns) for ri in range(NRINGS)]

    def body(x_ref, w_ref, sx_ref, sw_ref, out_ref,
             comm, send_sems, recv_sems, credit_sems):
        my = lax.axis_index("i")
        left = lax.rem(my + P - 1, P)
        right = lax.rem(my + 1, P)

        barrier = pltpu.get_barrier_semaphore()
        for nbr in (left, right):
            pl.semaphore_signal(barrier, inc=1, device_id=(nbr,),
                                device_id_type=pl.DeviceIdType.MESH)
        pl.semaphore_wait(barrier, 2)

        scale = sx_ref[0] * sw_ref[0]

        def dots(s):
            cR = lax.rem(my + 2 * P - 1 - s, P)
            cL = lax.rem(my + 1 + s, P)
            xR = x_ref[pl.ds(cR * m_chunk, m_chunk), :]
            xL = x_ref[pl.ds(cL * m_chunk, m_chunk), :]
            out = []
            for ri, (rightward, c0) in enumerate(ring_cfg):
                xc = xR if rightward else xL
                out.append(lax.dot_general(
                    xc, w_ref[:, c0:c0 + ns], (((1,), (0,)), ((), ())),
                    preferred_element_type=jnp.float32))
            return out

        def desc(ri, src_slot, dst_slot, dev):
            return pltpu.make_async_remote_copy(
                src_ref=comm.at[ri, src_slot],
                dst_ref=comm.at[ri, dst_slot],
                send_sem=send_sems.at[ri, src_slot],
                recv_sem=recv_sems.at[ri, dst_slot],
                device_id=(dev,),
                device_id_type=pl.DeviceIdType.MESH,
            )

        sends = [None] * NRINGS
        for s in range(P):
            slot = s % 2
            pd = dots(s)
            for ri, (rightward, c0) in enumerate(ring_cfg):
                dst = right if rightward else left
                src = left if rightward else right
                if s >= 1:
                    sends[ri].wait_send()
                    if 1 <= s - 1 <= P - 3:
                        pl.semaphore_signal(
                            credit_sems.at[ri], inc=1, device_id=(src,),
                            device_id_type=pl.DeviceIdType.MESH)
                    desc(ri, slot, slot, src).wait_recv()
                    acc = comm[ri, slot].astype(jnp.float32) + pd[ri]
                else:
                    acc = pd[ri]
                if s < P - 1:
                    comm[ri, slot] = acc.astype(COMM_DTYPE)
                    if s >= 2:
                        pl.semaphore_wait(credit_sems.at[ri], 1)
                    snd = desc(ri, slot, 1 - slot, dst)
                    snd.start()
                    sends[ri] = snd
                else:
                    out_ref[:, c0:c0 + ns] = acc * scale

    return pl.pallas_call(
        body,
        out_shape=jax.ShapeDtypeStruct((m_chunk, n), jnp.float32),
        in_specs=[
            pl.BlockSpec(memory_space=pltpu.VMEM),
            pl.BlockSpec(memory_space=pltpu.VMEM),
            pl.BlockSpec(memory_space=pltpu.SMEM),
            pl.BlockSpec(memory_space=pltpu.SMEM),
        ],
        out_specs=pl.BlockSpec(memory_space=pltpu.VMEM),
        scratch_shapes=[
            pltpu.VMEM((NRINGS, 2, m_chunk, ns), COMM_DTYPE),
            pltpu.SemaphoreType.DMA((NRINGS, 2)),
            pltpu.SemaphoreType.DMA((NRINGS, 2)),
            pltpu.SemaphoreType.REGULAR((NRINGS,)),
        ],
        compiler_params=pltpu.CompilerParams(collective_id=0),
    )(x, w_mat, scale_x, scale_w)


# device time: 358557 ns/iter; 1.0008x vs baseline; 1.0008x over previous
import jax
import jax.numpy as jnp
from jax import lax
from jax.experimental import pallas as pl
from jax.experimental.pallas import tpu as pltpu

P = 16
COMM_DTYPE = jnp.bfloat16
NRINGS = 4


def kernel(x, w_mat, scale_x, scale_w):
    m_full, _k_shard = x.shape
    _, n = w_mat.shape
    m_chunk = m_full // P
    ns = n // NRINGS
    ring_cfg = [(ri < NRINGS // 2, ri * ns) for ri in range(NRINGS)]

    def body(x_ref, w_ref, sx_ref, sw_ref, out_ref,
             comm, send_sems, recv_sems, credit_sems):
        my = lax.axis_index("i")
        left = lax.rem(my + P - 1, P)
        right = lax.rem(my + 1, P)

        barrier = pltpu.get_barrier_semaphore()
        for nbr in (left, right):
            pl.semaphore_signal(barrier, inc=1, device_id=(nbr,),
                                device_id_type=pl.DeviceIdType.MESH)
        pl.semaphore_wait(barrier, 2)

        scale = sx_ref[0] * sw_ref[0]

        def dots(s):
            cR = lax.rem(my + 2 * P - 1 - s, P)
            cL = lax.rem(my + 1 + s, P)
            xR = x_ref[pl.ds(cR * m_chunk, m_chunk), :]
            xL = x_ref[pl.ds(cL * m_chunk, m_chunk), :]
            out = []
            for ri, (rightward, c0) in enumerate(ring_cfg):
                xc = xR if rightward else xL
                pd = lax.dot_general(
                    xc, w_ref[:, c0:c0 + ns], (((1,), (0,)), ((), ())),
                    preferred_element_type=jnp.float32)
                out.append(pd if s == P - 1 else pd.astype(COMM_DTYPE))
            return out

        def desc(ri, src_slot, dst_slot, dev):
            return pltpu.make_async_remote_copy(
                src_ref=comm.at[ri, src_slot],
                dst_ref=comm.at[ri, dst_slot],
                send_sem=send_sems.at[ri, src_slot],
                recv_sem=recv_sems.at[ri, dst_slot],
                device_id=(dev,),
                device_id_type=pl.DeviceIdType.MESH,
            )

        sends = [None] * NRINGS
        for s in range(P):
            slot = s % 2
            pd = dots(s)
            for ri, (rightward, c0) in enumerate(ring_cfg):
                dst = right if rightward else left
                src = left if rightward else right
                if s >= 1:
                    sends[ri].wait_send()
                    if 1 <= s - 1 <= P - 3:
                        pl.semaphore_signal(
                            credit_sems.at[ri], inc=1, device_id=(src,),
                            device_id_type=pl.DeviceIdType.MESH)
                    desc(ri, slot, slot, src).wait_recv()
                    if s == P - 1:
                        acc = comm[ri, slot].astype(jnp.float32) + pd[ri]
                    else:
                        acc = comm[ri, slot] + pd[ri]
                else:
                    acc = pd[ri]
                if s < P - 1:
                    comm[ri, slot] = acc
                    if s >= 2:
                        pl.semaphore_wait(credit_sems.at[ri], 1)
                    snd = desc(ri, slot, 1 - slot, dst)
                    snd.start()
                    sends[ri] = snd
                else:
                    out_ref[:, c0:c0 + ns] = acc * scale

    return pl.pallas_call(
        body,
        out_shape=jax.ShapeDtypeStruct((m_chunk, n), jnp.float32),
        in_specs=[
            pl.BlockSpec(memory_space=pltpu.VMEM),
            pl.BlockSpec(memory_space=pltpu.VMEM),
            pl.BlockSpec(memory_space=pltpu.SMEM),
            pl.BlockSpec(memory_space=pltpu.SMEM),
        ],
        out_specs=pl.BlockSpec(memory_space=pltpu.VMEM),
        scratch_shapes=[
            pltpu.VMEM((NRINGS, 2, m_chunk, ns), COMM_DTYPE),
            pltpu.SemaphoreType.DMA((NRINGS, 2)),
            pltpu.SemaphoreType.DMA((NRINGS, 2)),
            pltpu.SemaphoreType.REGULAR((NRINGS,)),
        ],
        compiler_params=pltpu.CompilerParams(collective_id=0),
    )(x, w_mat, scale_x, scale_w)


# device time: 356611 ns/iter; 1.0062x vs baseline; 1.0055x over previous
import jax
import jax.numpy as jnp
from jax import lax
from jax.experimental import pallas as pl
from jax.experimental.pallas import tpu as pltpu

P = 16
COMM_DTYPE = jnp.bfloat16
NRINGS = 4


def kernel(x, w_mat, scale_x, scale_w):
    m_full, _k_shard = x.shape
    _, n = w_mat.shape
    m_chunk = m_full // P
    ns = n // NRINGS
    ring_cfg = [(True, 0), (True, ns), (False, 2 * ns), (False, 3 * ns)]

    def body(x_ref, w_ref, sx_ref, sw_ref, out_ref,
             comm, send_sems, recv_sems, credit_sems):
        my = lax.axis_index("i")
        left = lax.rem(my + P - 1, P)
        right = lax.rem(my + 1, P)

        barrier = pltpu.get_barrier_semaphore()
        for nbr in (left, right):
            pl.semaphore_signal(barrier, inc=1, device_id=(nbr,),
                                device_id_type=pl.DeviceIdType.MESH)
        pl.semaphore_wait(barrier, 2)

        scale = sx_ref[0] * sw_ref[0]

        def dots(s):
            cR = lax.rem(my + 2 * P - 1 - s, P)
            cL = lax.rem(my + 1 + s, P)
            xR = x_ref[pl.ds(cR * m_chunk, m_chunk), :]
            xL = x_ref[pl.ds(cL * m_chunk, m_chunk), :]
            out = []
            for ri, (rightward, c0) in enumerate(ring_cfg):
                xc = xR if rightward else xL
                out.append(lax.dot_general(
                    xc, w_ref[:, c0:c0 + ns], (((1,), (0,)), ((), ())),
                    preferred_element_type=jnp.float32))
            return out

        def desc(ri, src_slot, dst_slot, dev):
            return pltpu.make_async_remote_copy(
                src_ref=comm.at[ri, src_slot],
                dst_ref=comm.at[ri, dst_slot],
                send_sem=send_sems.at[ri, src_slot],
                recv_sem=recv_sems.at[ri, dst_slot],
                device_id=(dev,),
                device_id_type=pl.DeviceIdType.MESH,
            )

        sends = [None] * NRINGS
        for s in range(P):
            slot = s % 2
            pd = dots(s)
            for ri, (rightward, c0) in enumerate(ring_cfg):
                dst = right if rightward else left
                src = left if rightward else right
                if s >= 1:
                    sends[ri].wait_send()
                    if 1 <= s - 1 <= P - 3:
                        pl.semaphore_signal(
                            credit_sems.at[ri], inc=1, device_id=(src,),
                            device_id_type=pl.DeviceIdType.MESH)
                    desc(ri, slot, slot, src).wait_recv()
                    acc = comm[ri, slot].astype(jnp.float32) + pd[ri]
                else:
                    acc = pd[ri]
                if s < P - 1:
                    comm[ri, slot] = acc.astype(COMM_DTYPE)
                    if s >= 2:
                        pl.semaphore_wait(credit_sems.at[ri], 1)
                    snd = desc(ri, slot, 1 - slot, dst)
                    snd.start()
                    sends[ri] = snd
                else:
                    out_ref[:, c0:c0 + ns] = acc * scale

    return pl.pallas_call(
        body,
        out_shape=jax.ShapeDtypeStruct((m_chunk, n), jnp.float32),
        in_specs=[
            pl.BlockSpec(memory_space=pltpu.VMEM),
            pl.BlockSpec(memory_space=pltpu.VMEM),
            pl.BlockSpec(memory_space=pltpu.SMEM),
            pl.BlockSpec(memory_space=pltpu.SMEM),
        ],
        out_specs=pl.BlockSpec(memory_space=pltpu.VMEM),
        scratch_shapes=[
            pltpu.VMEM((NRINGS, 2, m_chunk, ns), COMM_DTYPE),
            pltpu.SemaphoreType.DMA((NRINGS, 2)),
            pltpu.SemaphoreType.DMA((NRINGS, 2)),
            pltpu.SemaphoreType.REGULAR((NRINGS,)),
        ],
        compiler_params=pltpu.CompilerParams(collective_id=0),
    )(x, w_mat, scale_x, scale_w)
